# Initial kernel scaffold; baseline (speedup 1.0000x reference)
import functools

import jax
import jax.numpy as jnp
from jax import lax
from jax.experimental import pallas as pl
from jax.experimental.pallas import tpu as pltpu
from jax.experimental.pallas import tpu_sc as plsc

F = 26
B = 16384
FIELD_DIM = 100000
L = 16
NC = 2
NS = 16
NW = NC * NS
PER_W = B // NW
E = PER_W * F
NCHUNK = PER_W // L


def _make_kernel():
    mesh = plsc.VectorSubcoreMesh(
        core_axis_name="c", subcore_axis_name="s", num_cores=NC, num_subcores=NS
    )

    @functools.partial(
        pl.kernel,
        mesh=mesh,
        out_type=jax.ShapeDtypeStruct((B,), jnp.float32),
        scratch_types=[
            pltpu.VMEM((E,), jnp.int32),
            pltpu.VMEM((E,), jnp.int32),
            pltpu.VMEM((E,), jnp.float32),
            pltpu.VMEM((PER_W,), jnp.float32),
            pltpu.VMEM((L,), jnp.float32),
            pltpu.SemaphoreType.DMA,
        ],
    )
    def k(x_hbm, table_hbm, bias_hbm, out_hbm, xv, idxv, rowsv, outv, biasv, sem):
        wid = lax.axis_index("s") * NC + lax.axis_index("c")
        base = wid * PER_W
        pltpu.sync_copy(x_hbm.at[pl.ds(base * F, E)], xv)
        pltpu.sync_copy(bias_hbm, biasv)

        lanes26 = lax.iota(jnp.int32, L) * F

        def build(c, carry):
            src_base = c * (L * F)
            for f in range(F):
                vals = plsc.load_gather(xv, [lanes26 + (src_base + f)])
                idxv[pl.ds(f * PER_W + c * L, L)] = vals + f * FIELD_DIM
            return carry

        lax.fori_loop(0, NCHUNK, build, 0)

        pltpu.async_copy(table_hbm.at[idxv], rowsv, sem).wait()

        bias_vec = biasv[...]

        def reduce(c, carry):
            acc = bias_vec
            for f in range(F):
                acc = acc + rowsv[pl.ds(f * PER_W + c * L, L)]
            outv[pl.ds(c * L, L)] = acc
            return carry

        lax.fori_loop(0, NCHUNK, reduce, 0)
        pltpu.sync_copy(outv, out_hbm.at[pl.ds(base, PER_W)])

    return k


_sc_kernel = _make_kernel()


@jax.jit
def kernel(x, table, bias):
    xf = x.reshape(-1)
    tf = table.reshape(-1)
    bb = jnp.broadcast_to(bias.reshape(-1)[:1], (L,))
    out = _sc_kernel(xf, tf, bb)
    return out.reshape(B, 1)


# speedup vs baseline: 1.1362x; 1.1362x over previous
import functools

import jax
import jax.numpy as jnp
from jax import lax
from jax.experimental import pallas as pl
from jax.experimental.pallas import tpu as pltpu
from jax.experimental.pallas import tpu_sc as plsc

F = 26
B = 16384
FIELD_DIM = 100000
L = 16
NC = 2
NS = 16
NW = NC * NS
PER_W = B // NW
E = PER_W * F
NCHUNK = PER_W // L


def _make_kernel():
    mesh = plsc.VectorSubcoreMesh(
        core_axis_name="c", subcore_axis_name="s", num_cores=NC, num_subcores=NS
    )

    @functools.partial(
        pl.kernel,
        mesh=mesh,
        out_type=jax.ShapeDtypeStruct((B,), jnp.float32),
        compiler_params=pltpu.CompilerParams(needs_layout_passes=False),
        scratch_types=[
            pltpu.VMEM((E,), jnp.int32),
            pltpu.VMEM((E,), jnp.int32),
            pltpu.VMEM((E,), jnp.float32),
            pltpu.VMEM((PER_W,), jnp.float32),
            pltpu.VMEM((L,), jnp.float32),
            pltpu.SemaphoreType.DMA,
        ],
    )
    def k(x_hbm, table_hbm, bias_hbm, out_hbm, xv, idxv, rowsv, outv, biasv, sem):
        wid = lax.axis_index("s") * NC + lax.axis_index("c")
        base = wid * PER_W
        pltpu.sync_copy(x_hbm.at[pl.ds(base * F, E)], xv)
        pltpu.sync_copy(bias_hbm, biasv)

        lanes26 = lax.iota(jnp.int32, L) * F

        def build(c, carry):
            src_base = c * (L * F)
            for f in range(F):
                vals = plsc.load_gather(xv, [lanes26 + (src_base + f)])
                idxv[pl.ds(f * PER_W + c * L, L)] = vals + f * FIELD_DIM
            return carry

        lax.fori_loop(0, NCHUNK, build, 0)

        pltpu.async_copy(table_hbm.at[idxv], rowsv, sem).wait()

        bias_vec = biasv[...]

        def reduce(c, carry):
            acc = bias_vec
            for f in range(F):
                acc = acc + rowsv[pl.ds(f * PER_W + c * L, L)]
            outv[pl.ds(c * L, L)] = acc
            return carry

        lax.fori_loop(0, NCHUNK, reduce, 0)
        pltpu.sync_copy(outv, out_hbm.at[pl.ds(base, PER_W)])

    return k


_sc_kernel = _make_kernel()


@jax.jit
def kernel(x, table, bias):
    xf = x.reshape(-1)
    tf = table.reshape(-1)
    bb = jnp.broadcast_to(bias.reshape(-1)[:1], (L,))
    out = _sc_kernel(xf, tf, bb)
    return out.reshape(B, 1)
